# speedup vs baseline: 15.6422x; 15.6422x over previous
import jax
import jax.numpy as jnp
from jax import lax
from jax.experimental import pallas as pl
from jax.experimental.pallas import tpu as pltpu
from jax.experimental.pallas import tpu_sc as plsc

N = 10000
D = 128
E = 320000

NC = 2
NS = 16
NW = NC * NS
EB = 128
ROWS_W = 80
EPAD = NW * ROWS_W * EB
NPAD = 10240
DEGW = 16
RPT = NPAD // NS
BLK = 640
GRID = NPAD // BLK

_mesh = plsc.VectorSubcoreMesh(core_axis_name="c", subcore_axis_name="s")


def _deg_body(dst_hbm, out_hbm, didx, vbuf, accum):
    c = lax.axis_index("c")
    s = lax.axis_index("s")
    wid = s * NC + c
    pltpu.sync_copy(dst_hbm.at[wid], didx)

    def _fill(val):
        def body(i, carry):
            vbuf[i, :] = jnp.full((DEGW,), val, jnp.float32)
            return carry
        lax.fori_loop(0, EB, body, 0)

    _fill(0.0)
    for k in range(RPT // EB):
        pltpu.sync_copy(vbuf, accum.at[pl.ds(s * RPT + k * EB, EB)])
    _fill(1.0)
    plsc.subcore_barrier()

    def body(j, carry):
        pltpu.sync_copy(vbuf, accum.at[didx.at[j]], add=True)
        return carry
    lax.fori_loop(0, ROWS_W, body, 0)
    plsc.subcore_barrier()
    pltpu.sync_copy(accum.at[pl.ds(s * RPT, RPT)],
                    out_hbm.at[pl.ds(c * NPAD + s * RPT, RPT)])


_deg_call = pl.kernel(
    _deg_body,
    out_type=jax.ShapeDtypeStruct((NC * NPAD, DEGW), jnp.float32),
    mesh=_mesh,
    scratch_types=[
        pltpu.VMEM((ROWS_W, EB), jnp.int32),
        pltpu.VMEM((EB, DEGW), jnp.float32),
        pltpu.VMEM_SHARED((NPAD, DEGW), jnp.float32),
    ],
)


def _agg_body(src_hbm, dst_hbm, u_hbm, out_hbm, sidx, didx, rows, accum, sem):
    c = lax.axis_index("c")
    s = lax.axis_index("s")
    wid = s * NC + c
    pltpu.sync_copy(src_hbm.at[wid], sidx)
    pltpu.sync_copy(dst_hbm.at[wid], didx)

    def zbody(i, carry):
        rows[i // 8, pl.ds((i % 8) * 16, 16)] = jnp.zeros((16,), jnp.float32)
        return carry
    lax.fori_loop(0, EB * 8, zbody, 0)
    for k in range(RPT // EB):
        pltpu.sync_copy(rows, accum.at[pl.ds(s * RPT + k * EB, EB)])
    plsc.subcore_barrier()

    def body(j, carry):
        pltpu.async_copy(u_hbm.at[sidx.at[j]], rows, sem).wait()
        pltpu.sync_copy(rows, accum.at[didx.at[j]], add=True)
        return carry
    lax.fori_loop(0, ROWS_W, body, 0)
    plsc.subcore_barrier()
    pltpu.sync_copy(accum.at[pl.ds(s * RPT, RPT)],
                    out_hbm.at[pl.ds(c * NPAD + s * RPT, RPT)])


_agg_call = pl.kernel(
    _agg_body,
    out_type=jax.ShapeDtypeStruct((NC * NPAD, D), jnp.float32),
    mesh=_mesh,
    scratch_types=[
        pltpu.VMEM((ROWS_W, EB), jnp.int32),
        pltpu.VMEM((ROWS_W, EB), jnp.int32),
        pltpu.VMEM((EB, D), jnp.float32),
        pltpu.VMEM_SHARED((NPAD, D), jnp.float32),
        pltpu.SemaphoreType.DMA,
    ],
)


def _prep_body(degp_ref, x_ref, dinv_ref, u0_ref):
    i = pl.program_id(0)
    deg = degp_ref[0, :, 0:1] + degp_ref[1, :, 0:1] + 1.0
    rows = i * BLK + lax.broadcasted_iota(jnp.int32, (BLK, 1), 0)
    dinv = jnp.where(rows < N, lax.rsqrt(deg), 0.0)
    dinv_ref[...] = dinv
    u0_ref[...] = x_ref[...] * dinv


_prep_call = pl.pallas_call(
    _prep_body,
    grid=(GRID,),
    in_specs=[
        pl.BlockSpec((NC, BLK, DEGW), lambda i: (0, i, 0)),
        pl.BlockSpec((BLK, D), lambda i: (i, 0)),
    ],
    out_specs=[
        pl.BlockSpec((BLK, 1), lambda i: (i, 0)),
        pl.BlockSpec((BLK, D), lambda i: (i, 0)),
    ],
    out_shape=[
        jax.ShapeDtypeStruct((NPAD, 1), jnp.float32),
        jax.ShapeDtypeStruct((NPAD, D), jnp.float32),
    ],
)


def _layer_body(p_ref, u_ref, dinv_ref, w_ref, b_ref, unext_ref):
    sm = p_ref[0] + p_ref[1] + u_ref[...]
    t = jnp.dot(sm, w_ref[...], preferred_element_type=jnp.float32)
    o = dinv_ref[...] * t + b_ref[...]
    unext_ref[...] = dinv_ref[...] * jnp.maximum(o, 0.0)


_layer_call = pl.pallas_call(
    _layer_body,
    grid=(GRID,),
    in_specs=[
        pl.BlockSpec((NC, BLK, D), lambda i: (0, i, 0)),
        pl.BlockSpec((BLK, D), lambda i: (i, 0)),
        pl.BlockSpec((BLK, 1), lambda i: (i, 0)),
        pl.BlockSpec((D, D), lambda i: (0, 0)),
        pl.BlockSpec((1, D), lambda i: (0, 0)),
    ],
    out_specs=pl.BlockSpec((BLK, D), lambda i: (i, 0)),
    out_shape=jax.ShapeDtypeStruct((NPAD, D), jnp.float32),
)


def _final_body(p_ref, u_ref, dinv_ref, w_ref, b_ref, wl_ref, bl_ref,
                gsum_ref, res_ref):
    i = pl.program_id(0)
    sm = p_ref[0] + p_ref[1] + u_ref[...]
    t = jnp.dot(sm, w_ref[...], preferred_element_type=jnp.float32)
    o = dinv_ref[...] * t + b_ref[...]
    o = jnp.where(dinv_ref[...] > 0.0, o, 0.0)
    part = jnp.sum(o, axis=0, keepdims=True)

    @pl.when(i == 0)
    def _():
        gsum_ref[...] = part

    @pl.when(i > 0)
    def _():
        gsum_ref[...] += part

    @pl.when(i == pl.num_programs(0) - 1)
    def _():
        g = gsum_ref[...] * (1.0 / N)
        z = jnp.dot(g, wl_ref[...], preferred_element_type=jnp.float32)
        res_ref[...] = jax.nn.sigmoid(z + bl_ref[...])


_final_call = pl.pallas_call(
    _final_body,
    grid=(GRID,),
    in_specs=[
        pl.BlockSpec((NC, BLK, D), lambda i: (0, i, 0)),
        pl.BlockSpec((BLK, D), lambda i: (i, 0)),
        pl.BlockSpec((BLK, 1), lambda i: (i, 0)),
        pl.BlockSpec((D, D), lambda i: (0, 0)),
        pl.BlockSpec((1, D), lambda i: (0, 0)),
        pl.BlockSpec((D, 1), lambda i: (0, 0)),
        pl.BlockSpec((1, 1), lambda i: (0, 0)),
    ],
    out_specs=[
        pl.BlockSpec((1, D), lambda i: (0, 0)),
        pl.BlockSpec((1, 1), lambda i: (0, 0)),
    ],
    out_shape=[
        jax.ShapeDtypeStruct((1, D), jnp.float32),
        jax.ShapeDtypeStruct((1, 1), jnp.float32),
    ],
)


def kernel(x, edge_index, pos, W1, b1, W2, b2, W3, b3, Wl, bl):
    del pos
    src = edge_index[0]
    dst = edge_index[1]
    npad_e = EPAD - E
    padv = (N + (jnp.arange(npad_e, dtype=jnp.int32) % (NPAD - N)))
    srcp = jnp.concatenate([src, padv]).reshape(NW, ROWS_W, EB)
    dstp = jnp.concatenate([dst, padv]).reshape(NW, ROWS_W, EB)
    xp = jnp.pad(x, ((0, NPAD - N), (0, 0)))

    degp = _deg_call(dstp).reshape(NC, NPAD, DEGW)
    dinv, u = _prep_call(degp, xp)
    for (W, b) in ((W1, b1), (W2, b2)):
        P = _agg_call(srcp, dstp, u).reshape(NC, NPAD, D)
        u = _layer_call(P, u, dinv, W, b.reshape(1, D))
    P = _agg_call(srcp, dstp, u).reshape(NC, NPAD, D)
    _, res = _final_call(P, u, dinv, W3, b3.reshape(1, D),
                         Wl, bl.reshape(1, 1))
    return res


# speedup vs baseline: 21.2872x; 1.3609x over previous
import jax
import jax.numpy as jnp
from jax import lax
from jax.experimental import pallas as pl
from jax.experimental.pallas import tpu as pltpu
from jax.experimental.pallas import tpu_sc as plsc

N = 10000
D = 128
E = 320000

NC = 2
NS = 16
NW = NC * NS
EB = 128
ROWS_W = 80
CR = 16
NCHUNK = ROWS_W // CR
NACC = 10112
RPTA = NACC // NS
EPAD = NW * ROWS_W * EB
NPAD = 10240
DEGW = 16
RPT = NPAD // NS
BLK = 640
GRID = NPAD // BLK

_mesh = plsc.VectorSubcoreMesh(core_axis_name="c", subcore_axis_name="s")


def _deg_body(dst_hbm, out_hbm, didx, vbuf, accum):
    c = lax.axis_index("c")
    s = lax.axis_index("s")
    wid = s * NC + c
    pltpu.sync_copy(dst_hbm.at[pl.ds(wid * ROWS_W, ROWS_W)], didx)

    def _fill(val):
        def body(i, carry):
            vbuf[i, :] = jnp.full((DEGW,), val, jnp.float32)
            return carry
        lax.fori_loop(0, EB, body, 0)

    _fill(0.0)
    for k in range(RPT // EB):
        pltpu.sync_copy(vbuf, accum.at[pl.ds(s * RPT + k * EB, EB)])
    _fill(1.0)
    plsc.subcore_barrier()

    def body(j, carry):
        pltpu.sync_copy(vbuf, accum.at[didx.at[j]], add=True)
        return carry
    lax.fori_loop(0, ROWS_W, body, 0)
    plsc.subcore_barrier()
    pltpu.sync_copy(accum.at[pl.ds(s * RPT, RPT)],
                    out_hbm.at[pl.ds(c * NPAD + s * RPT, RPT)])


_deg_call = pl.kernel(
    _deg_body,
    out_type=jax.ShapeDtypeStruct((NC * NPAD, DEGW), jnp.float32),
    mesh=_mesh,
    scratch_types=[
        pltpu.VMEM((ROWS_W, EB), jnp.int32),
        pltpu.VMEM((EB, DEGW), jnp.float32),
        pltpu.VMEM_SHARED((NPAD, DEGW), jnp.float32),
    ],
)


def _agg_body(src_hbm, dst_hbm, u_hbm, out_hbm, sidx, didx, rows0, rows1,
              accum, gsem0, gsem1):
    c = lax.axis_index("c")
    s = lax.axis_index("s")
    wid = s * NC + c

    def zero_rows0(nrows):
        def zbody(i, carry):
            rows0[i // 8, pl.ds((i % 8) * 16, 16)] = jnp.zeros((16,),
                                                              jnp.float32)
            return carry
        lax.fori_loop(0, nrows * (D // 16), zbody, 0)

    zero_rows0(EB)
    for k in range(RPTA // EB):
        pltpu.sync_copy(rows0, accum.at[pl.ds(s * RPTA + k * EB, EB)])
    rem = RPTA % EB
    pltpu.sync_copy(rows0.at[pl.ds(0, rem)],
                    accum.at[pl.ds(s * RPTA + (RPTA // EB) * EB, rem)])
    plsc.subcore_barrier()

    def chunk_body(ci, carry):
        base = wid * ROWS_W + ci * CR
        pltpu.sync_copy(src_hbm.at[pl.ds(base, CR)], sidx)
        pltpu.sync_copy(dst_hbm.at[pl.ds(base, CR)], didx)
        pltpu.async_copy(u_hbm.at[sidx.at[0]], rows0, gsem0)

        def body(g, c2):
            j0 = 2 * g
            pltpu.async_copy(u_hbm.at[sidx.at[j0 + 1]], rows1, gsem1)
            pltpu.make_async_copy(u_hbm.at[sidx.at[j0]], rows0, gsem0).wait()
            pltpu.sync_copy(rows0, accum.at[didx.at[j0]], add=True)

            @pl.when(j0 + 2 < CR)
            def _():
                pltpu.async_copy(u_hbm.at[sidx.at[j0 + 2]], rows0, gsem0)

            pltpu.make_async_copy(u_hbm.at[sidx.at[j0 + 1]], rows1,
                                  gsem1).wait()
            pltpu.sync_copy(rows1, accum.at[didx.at[j0 + 1]], add=True)
            return c2
        lax.fori_loop(0, CR // 2, body, 0)
        return carry
    lax.fori_loop(0, NCHUNK, chunk_body, 0)
    plsc.subcore_barrier()
    pltpu.sync_copy(accum.at[pl.ds(s * RPTA, RPTA)],
                    out_hbm.at[pl.ds(c * NPAD + s * RPTA, RPTA)])
    tail = (NPAD - NACC) // NS
    zero_rows0(tail)
    pltpu.sync_copy(rows0.at[pl.ds(0, tail)],
                    out_hbm.at[pl.ds(c * NPAD + NACC + s * tail, tail)])


_agg_call = pl.kernel(
    _agg_body,
    out_type=jax.ShapeDtypeStruct((NC * NPAD, D), jnp.float32),
    mesh=_mesh,
    scratch_types=[
        pltpu.VMEM((CR, EB), jnp.int32),
        pltpu.VMEM((CR, EB), jnp.int32),
        pltpu.VMEM((EB, D), jnp.float32),
        pltpu.VMEM((EB, D), jnp.float32),
        pltpu.VMEM_SHARED((NACC, D), jnp.float32),
        pltpu.SemaphoreType.DMA,
        pltpu.SemaphoreType.DMA,
    ],
)


def _prep_body(degp_ref, x_ref, dinv_ref, u0_ref):
    i = pl.program_id(0)
    deg = degp_ref[0, :, 0:1] + degp_ref[1, :, 0:1] + 1.0
    rows = i * BLK + lax.broadcasted_iota(jnp.int32, (BLK, 1), 0)
    dinv = jnp.where(rows < N, lax.rsqrt(deg), 0.0)
    dinv_ref[...] = dinv
    u0_ref[...] = x_ref[...] * dinv


_prep_call = pl.pallas_call(
    _prep_body,
    grid=(GRID,),
    in_specs=[
        pl.BlockSpec((NC, BLK, DEGW), lambda i: (0, i, 0)),
        pl.BlockSpec((BLK, D), lambda i: (i, 0)),
    ],
    out_specs=[
        pl.BlockSpec((BLK, 1), lambda i: (i, 0)),
        pl.BlockSpec((BLK, D), lambda i: (i, 0)),
    ],
    out_shape=[
        jax.ShapeDtypeStruct((NPAD, 1), jnp.float32),
        jax.ShapeDtypeStruct((NPAD, D), jnp.float32),
    ],
)


def _layer_body(p_ref, u_ref, dinv_ref, w_ref, b_ref, unext_ref):
    sm = p_ref[0] + p_ref[1] + u_ref[...]
    t = jnp.dot(sm, w_ref[...], preferred_element_type=jnp.float32)
    o = dinv_ref[...] * t + b_ref[...]
    unext_ref[...] = dinv_ref[...] * jnp.maximum(o, 0.0)


_layer_call = pl.pallas_call(
    _layer_body,
    grid=(GRID,),
    in_specs=[
        pl.BlockSpec((NC, BLK, D), lambda i: (0, i, 0)),
        pl.BlockSpec((BLK, D), lambda i: (i, 0)),
        pl.BlockSpec((BLK, 1), lambda i: (i, 0)),
        pl.BlockSpec((D, D), lambda i: (0, 0)),
        pl.BlockSpec((1, D), lambda i: (0, 0)),
    ],
    out_specs=pl.BlockSpec((BLK, D), lambda i: (i, 0)),
    out_shape=jax.ShapeDtypeStruct((NPAD, D), jnp.float32),
)


def _final_body(p_ref, u_ref, dinv_ref, w_ref, b_ref, wl_ref, bl_ref,
                gsum_ref, res_ref):
    i = pl.program_id(0)
    sm = p_ref[0] + p_ref[1] + u_ref[...]
    t = jnp.dot(sm, w_ref[...], preferred_element_type=jnp.float32)
    o = dinv_ref[...] * t + b_ref[...]
    o = jnp.where(dinv_ref[...] > 0.0, o, 0.0)
    part = jnp.sum(o, axis=0, keepdims=True)

    @pl.when(i == 0)
    def _():
        gsum_ref[...] = part

    @pl.when(i > 0)
    def _():
        gsum_ref[...] += part

    @pl.when(i == pl.num_programs(0) - 1)
    def _():
        g = gsum_ref[...] * (1.0 / N)
        z = jnp.dot(g, wl_ref[...], preferred_element_type=jnp.float32)
        res_ref[...] = jax.nn.sigmoid(z + bl_ref[...])


_final_call = pl.pallas_call(
    _final_body,
    grid=(GRID,),
    in_specs=[
        pl.BlockSpec((NC, BLK, D), lambda i: (0, i, 0)),
        pl.BlockSpec((BLK, D), lambda i: (i, 0)),
        pl.BlockSpec((BLK, 1), lambda i: (i, 0)),
        pl.BlockSpec((D, D), lambda i: (0, 0)),
        pl.BlockSpec((1, D), lambda i: (0, 0)),
        pl.BlockSpec((D, 1), lambda i: (0, 0)),
        pl.BlockSpec((1, 1), lambda i: (0, 0)),
    ],
    out_specs=[
        pl.BlockSpec((1, D), lambda i: (0, 0)),
        pl.BlockSpec((1, 1), lambda i: (0, 0)),
    ],
    out_shape=[
        jax.ShapeDtypeStruct((1, D), jnp.float32),
        jax.ShapeDtypeStruct((1, 1), jnp.float32),
    ],
)


def kernel(x, edge_index, pos, W1, b1, W2, b2, W3, b3, Wl, bl):
    del pos
    src = edge_index[0]
    dst = edge_index[1]
    npad_e = EPAD - E
    pads = (N + (jnp.arange(npad_e, dtype=jnp.int32) % (NPAD - N)))
    padd = (N + (jnp.arange(npad_e, dtype=jnp.int32) % (NACC - N)))
    srcp = jnp.concatenate([src, pads]).reshape(NW * ROWS_W, EB)
    dstp = jnp.concatenate([dst, padd]).reshape(NW * ROWS_W, EB)
    xp = jnp.pad(x, ((0, NPAD - N), (0, 0)))

    degp = _deg_call(dstp).reshape(NC, NPAD, DEGW)
    dinv, u = _prep_call(degp, xp)
    for (W, b) in ((W1, b1), (W2, b2)):
        P = _agg_call(srcp, dstp, u).reshape(NC, NPAD, D)
        u = _layer_call(P, u, dinv, W, b.reshape(1, D))
    P = _agg_call(srcp, dstp, u).reshape(NC, NPAD, D)
    _, res = _final_call(P, u, dinv, W3, b3.reshape(1, D),
                         Wl, bl.reshape(1, 1))
    return res


# speedup vs baseline: 22.6591x; 1.0644x over previous
import jax
import jax.numpy as jnp
from jax import lax
from jax.experimental import pallas as pl
from jax.experimental.pallas import tpu as pltpu
from jax.experimental.pallas import tpu_sc as plsc

N = 10000
D = 128
E = 320000

NC = 2
NS = 16
NW = NC * NS
EB = 128
ROWS_W = 80
CR = 16
NCHUNK = ROWS_W // CR
NACC = 10112
RPTA = NACC // NS
EPAD = NW * ROWS_W * EB
NPAD = 10240
DEGW = 16
RPT = NPAD // NS
BLK = 640
GRID = NPAD // BLK

_mesh = plsc.VectorSubcoreMesh(core_axis_name="c", subcore_axis_name="s")


def _deg_body(dst_hbm, out_hbm, didx, vbuf, accum):
    c = lax.axis_index("c")
    s = lax.axis_index("s")
    wid = s * NC + c
    pltpu.sync_copy(dst_hbm.at[pl.ds(wid * ROWS_W, ROWS_W)], didx)

    def _fill(val):
        def body(i, carry):
            vbuf[i, :] = jnp.full((DEGW,), val, jnp.float32)
            return carry
        lax.fori_loop(0, EB, body, 0)

    _fill(0.0)
    for k in range(RPT // EB):
        pltpu.sync_copy(vbuf, accum.at[pl.ds(s * RPT + k * EB, EB)])
    _fill(1.0)
    plsc.subcore_barrier()

    def body(j, carry):
        pltpu.sync_copy(vbuf, accum.at[didx.at[j]], add=True)
        return carry
    lax.fori_loop(0, ROWS_W, body, 0)
    plsc.subcore_barrier()
    pltpu.sync_copy(accum.at[pl.ds(s * RPT, RPT)],
                    out_hbm.at[pl.ds(c * NPAD + s * RPT, RPT)])


_deg_call = pl.kernel(
    _deg_body,
    out_type=jax.ShapeDtypeStruct((NC * NPAD, DEGW), jnp.float32),
    mesh=_mesh,
    scratch_types=[
        pltpu.VMEM((ROWS_W, EB), jnp.int32),
        pltpu.VMEM((EB, DEGW), jnp.float32),
        pltpu.VMEM_SHARED((NPAD, DEGW), jnp.float32),
    ],
)


def _agg_body(src_hbm, dst_hbm, u_hbm, out_hbm, sidxA, didxA, sidxB, didxB,
              rows0, rows1, accum, gsem0, gsem1, isem0, isem1):
    c = lax.axis_index("c")
    s = lax.axis_index("s")
    wid = s * NC + c

    def zero_rows0(nrows):
        def zbody(i, carry):
            rows0[i // 8, pl.ds((i % 8) * 16, 16)] = jnp.zeros((16,),
                                                              jnp.float32)
            return carry
        lax.fori_loop(0, nrows * (D // 16), zbody, 0)

    zero_rows0(EB)
    for k in range(RPTA // EB):
        pltpu.sync_copy(rows0, accum.at[pl.ds(s * RPTA + k * EB, EB)])
    rem = RPTA % EB
    pltpu.sync_copy(rows0.at[pl.ds(0, rem)],
                    accum.at[pl.ds(s * RPTA + (RPTA // EB) * EB, rem)])
    plsc.subcore_barrier()

    base0 = wid * ROWS_W
    ibufs = ((sidxA, didxA), (sidxB, didxB))
    pltpu.sync_copy(src_hbm.at[pl.ds(base0, CR)], sidxA)
    pltpu.sync_copy(dst_hbm.at[pl.ds(base0, CR)], didxA)
    pltpu.async_copy(u_hbm.at[sidxA.at[0]], rows0, gsem0)

    for ci in range(NCHUNK):
        sA, dA = ibufs[ci % 2]
        sB, dB = ibufs[(ci + 1) % 2]
        nbase = base0 + (ci + 1) * CR
        if ci + 1 < NCHUNK:
            pltpu.async_copy(src_hbm.at[pl.ds(nbase, CR)], sB, isem0)
            pltpu.async_copy(dst_hbm.at[pl.ds(nbase, CR)], dB, isem1)

        def body(g, c2, sA=sA, dA=dA, sB=sB, dB=dB, ci=ci, nbase=nbase):
            j0 = 2 * g
            pltpu.async_copy(u_hbm.at[sA.at[j0 + 1]], rows1, gsem1)
            pltpu.make_async_copy(u_hbm.at[sA.at[j0]], rows0, gsem0).wait()
            pltpu.sync_copy(rows0, accum.at[dA.at[j0]], add=True)

            if ci + 1 < NCHUNK:
                @pl.when(j0 + 2 < CR)
                def _():
                    pltpu.async_copy(u_hbm.at[sA.at[j0 + 2]], rows0, gsem0)

                @pl.when(j0 + 2 >= CR)
                def _():
                    pltpu.make_async_copy(src_hbm.at[pl.ds(nbase, CR)], sB,
                                          isem0).wait()
                    pltpu.make_async_copy(dst_hbm.at[pl.ds(nbase, CR)], dB,
                                          isem1).wait()
                    pltpu.async_copy(u_hbm.at[sB.at[0]], rows0, gsem0)
            else:
                @pl.when(j0 + 2 < CR)
                def _():
                    pltpu.async_copy(u_hbm.at[sA.at[j0 + 2]], rows0, gsem0)

            pltpu.make_async_copy(u_hbm.at[sA.at[j0 + 1]], rows1,
                                  gsem1).wait()
            pltpu.sync_copy(rows1, accum.at[dA.at[j0 + 1]], add=True)
            return c2
        lax.fori_loop(0, CR // 2, body, 0)
    plsc.subcore_barrier()
    pltpu.sync_copy(accum.at[pl.ds(s * RPTA, RPTA)],
                    out_hbm.at[pl.ds(c * NPAD + s * RPTA, RPTA)])
    tail = (NPAD - NACC) // NS
    zero_rows0(tail)
    pltpu.sync_copy(rows0.at[pl.ds(0, tail)],
                    out_hbm.at[pl.ds(c * NPAD + NACC + s * tail, tail)])


_agg_call = pl.kernel(
    _agg_body,
    out_type=jax.ShapeDtypeStruct((NC * NPAD, D), jnp.float32),
    mesh=_mesh,
    scratch_types=[
        pltpu.VMEM((CR, EB), jnp.int32),
        pltpu.VMEM((CR, EB), jnp.int32),
        pltpu.VMEM((CR, EB), jnp.int32),
        pltpu.VMEM((CR, EB), jnp.int32),
        pltpu.VMEM((EB, D), jnp.float32),
        pltpu.VMEM((EB, D), jnp.float32),
        pltpu.VMEM_SHARED((NACC, D), jnp.float32),
        pltpu.SemaphoreType.DMA,
        pltpu.SemaphoreType.DMA,
        pltpu.SemaphoreType.DMA,
        pltpu.SemaphoreType.DMA,
    ],
)


def _prep_body(degp_ref, x_ref, dinv_ref, u0_ref):
    i = pl.program_id(0)
    deg = degp_ref[0, :, 0:1] + degp_ref[1, :, 0:1] + 1.0
    rows = i * BLK + lax.broadcasted_iota(jnp.int32, (BLK, 1), 0)
    dinv = jnp.where(rows < N, lax.rsqrt(deg), 0.0)
    dinv_ref[...] = dinv
    u0_ref[...] = x_ref[...] * dinv


_prep_call = pl.pallas_call(
    _prep_body,
    grid=(GRID,),
    in_specs=[
        pl.BlockSpec((NC, BLK, DEGW), lambda i: (0, i, 0)),
        pl.BlockSpec((BLK, D), lambda i: (i, 0)),
    ],
    out_specs=[
        pl.BlockSpec((BLK, 1), lambda i: (i, 0)),
        pl.BlockSpec((BLK, D), lambda i: (i, 0)),
    ],
    out_shape=[
        jax.ShapeDtypeStruct((NPAD, 1), jnp.float32),
        jax.ShapeDtypeStruct((NPAD, D), jnp.float32),
    ],
)


def _layer_body(p_ref, u_ref, dinv_ref, w_ref, b_ref, unext_ref):
    sm = p_ref[0] + p_ref[1] + u_ref[...]
    t = jnp.dot(sm, w_ref[...], preferred_element_type=jnp.float32)
    o = dinv_ref[...] * t + b_ref[...]
    unext_ref[...] = dinv_ref[...] * jnp.maximum(o, 0.0)


_layer_call = pl.pallas_call(
    _layer_body,
    grid=(GRID,),
    in_specs=[
        pl.BlockSpec((NC, BLK, D), lambda i: (0, i, 0)),
        pl.BlockSpec((BLK, D), lambda i: (i, 0)),
        pl.BlockSpec((BLK, 1), lambda i: (i, 0)),
        pl.BlockSpec((D, D), lambda i: (0, 0)),
        pl.BlockSpec((1, D), lambda i: (0, 0)),
    ],
    out_specs=pl.BlockSpec((BLK, D), lambda i: (i, 0)),
    out_shape=jax.ShapeDtypeStruct((NPAD, D), jnp.float32),
)


def _final_body(p_ref, u_ref, dinv_ref, w_ref, b_ref, wl_ref, bl_ref,
                gsum_ref, res_ref):
    i = pl.program_id(0)
    sm = p_ref[0] + p_ref[1] + u_ref[...]
    t = jnp.dot(sm, w_ref[...], preferred_element_type=jnp.float32)
    o = dinv_ref[...] * t + b_ref[...]
    o = jnp.where(dinv_ref[...] > 0.0, o, 0.0)
    part = jnp.sum(o, axis=0, keepdims=True)

    @pl.when(i == 0)
    def _():
        gsum_ref[...] = part

    @pl.when(i > 0)
    def _():
        gsum_ref[...] += part

    @pl.when(i == pl.num_programs(0) - 1)
    def _():
        g = gsum_ref[...] * (1.0 / N)
        z = jnp.dot(g, wl_ref[...], preferred_element_type=jnp.float32)
        res_ref[...] = jax.nn.sigmoid(z + bl_ref[...])


_final_call = pl.pallas_call(
    _final_body,
    grid=(GRID,),
    in_specs=[
        pl.BlockSpec((NC, BLK, D), lambda i: (0, i, 0)),
        pl.BlockSpec((BLK, D), lambda i: (i, 0)),
        pl.BlockSpec((BLK, 1), lambda i: (i, 0)),
        pl.BlockSpec((D, D), lambda i: (0, 0)),
        pl.BlockSpec((1, D), lambda i: (0, 0)),
        pl.BlockSpec((D, 1), lambda i: (0, 0)),
        pl.BlockSpec((1, 1), lambda i: (0, 0)),
    ],
    out_specs=[
        pl.BlockSpec((1, D), lambda i: (0, 0)),
        pl.BlockSpec((1, 1), lambda i: (0, 0)),
    ],
    out_shape=[
        jax.ShapeDtypeStruct((1, D), jnp.float32),
        jax.ShapeDtypeStruct((1, 1), jnp.float32),
    ],
)


def kernel(x, edge_index, pos, W1, b1, W2, b2, W3, b3, Wl, bl):
    del pos
    src = edge_index[0]
    dst = edge_index[1]
    npad_e = EPAD - E
    pads = (N + (jnp.arange(npad_e, dtype=jnp.int32) % (NPAD - N)))
    padd = (N + (jnp.arange(npad_e, dtype=jnp.int32) % (NACC - N)))
    srcp = jnp.concatenate([src, pads]).reshape(NW * ROWS_W, EB)
    dstp = jnp.concatenate([dst, padd]).reshape(NW * ROWS_W, EB)
    xp = jnp.pad(x, ((0, NPAD - N), (0, 0)))

    degp = _deg_call(dstp).reshape(NC, NPAD, DEGW)
    dinv, u = _prep_call(degp, xp)
    for (W, b) in ((W1, b1), (W2, b2)):
        P = _agg_call(srcp, dstp, u).reshape(NC, NPAD, D)
        u = _layer_call(P, u, dinv, W, b.reshape(1, D))
    P = _agg_call(srcp, dstp, u).reshape(NC, NPAD, D)
    _, res = _final_call(P, u, dinv, W3, b3.reshape(1, D),
                         Wl, bl.reshape(1, 1))
    return res
